# baseline (device time: 13617 ns/iter reference)
import jax
import jax.numpy as jnp
from jax import lax
from jax.experimental import pallas as pl
from jax.experimental.pallas import tpu as pltpu

_NC = 4


def kernel(x, gamma):
    m, n_local = x.shape
    n_global = 2 * n_local
    ch = m // _NC
    gamma2 = gamma.reshape(1, n_local)

    def body(x_hbm, g_ref, o_hbm, x_vmem, out_vmem, send_buf, recv_buf,
             in_sems, out_sems, send_sem, recv_sem):
        my_x = lax.axis_index("x")
        my_y = lax.axis_index("y")
        nbr = (my_x, 1 - my_y)

        barrier_sem = pltpu.get_barrier_semaphore()
        pl.semaphore_signal(
            barrier_sem, inc=1,
            device_id=nbr, device_id_type=pl.DeviceIdType.MESH,
        )

        copies = []
        for k in range(_NC):
            rows = pl.ds(k * ch, ch)
            cp = pltpu.make_async_copy(
                x_hbm.at[rows, :], x_vmem.at[rows, :], in_sems.at[k]
            )
            cp.start()
            copies.append(cp)
        for k in range(_NC):
            rows = pl.ds(k * ch, ch)
            copies[k].wait()
            xc = x_vmem[rows, :]
            send_buf[rows, :] = jnp.sum(xc * xc, axis=1, keepdims=True)

        pl.semaphore_wait(barrier_sem, 1)
        rdma = pltpu.make_async_remote_copy(
            src_ref=send_buf,
            dst_ref=recv_buf,
            send_sem=send_sem,
            recv_sem=recv_sem,
            device_id=nbr,
            device_id_type=pl.DeviceIdType.MESH,
        )
        rdma.start()
        rdma.wait()

        g = g_ref[...]

        out_copies = []
        for k in range(_NC):
            rows = pl.ds(k * ch, ch)
            xc = x_vmem[rows, :]
            totalc = send_buf[rows, :] + recv_buf[rows, :]
            invc = lax.rsqrt(totalc / n_global + 1e-5)
            out_vmem[rows, :] = (xc * g * invc).astype(jnp.bfloat16)
            cp = pltpu.make_async_copy(
                out_vmem.at[rows, :], o_hbm.at[rows, :], out_sems.at[k]
            )
            cp.start()
            out_copies.append(cp)
        for cp in out_copies:
            cp.wait()

    return pl.pallas_call(
        body,
        out_shape=jax.ShapeDtypeStruct((m, n_local), jnp.bfloat16),
        in_specs=[
            pl.BlockSpec(memory_space=pl.ANY),
            pl.BlockSpec(memory_space=pltpu.VMEM),
        ],
        out_specs=pl.BlockSpec(memory_space=pl.ANY),
        scratch_shapes=[
            pltpu.VMEM((m, n_local), jnp.float32),
            pltpu.VMEM((m, n_local), jnp.bfloat16),
            pltpu.VMEM((m, 1), jnp.float32),
            pltpu.VMEM((m, 1), jnp.float32),
            pltpu.SemaphoreType.DMA((_NC,)),
            pltpu.SemaphoreType.DMA((_NC,)),
            pltpu.SemaphoreType.DMA,
            pltpu.SemaphoreType.DMA,
        ],
        compiler_params=pltpu.CompilerParams(collective_id=0),
    )(x, gamma2)


# device time: 12694 ns/iter; 1.0727x vs baseline; 1.0727x over previous
import jax
import jax.numpy as jnp
from jax import lax
from jax.experimental import pallas as pl
from jax.experimental.pallas import tpu as pltpu

_NC = 4


def kernel(x, gamma):
    m, n_local = x.shape
    n_global = 2 * n_local
    ch = m // _NC
    gamma2 = gamma.reshape(1, n_local)

    def body(x_ref, g_ref, o_ref, send_buf, recv_buf, send_sems, recv_sems):
        my_x = lax.axis_index("x")
        my_y = lax.axis_index("y")
        nbr = (my_x, 1 - my_y)

        barrier_sem = pltpu.get_barrier_semaphore()
        pl.semaphore_signal(
            barrier_sem, inc=1,
            device_id=nbr, device_id_type=pl.DeviceIdType.MESH,
        )

        rdmas = []
        for k in range(_NC):
            rows = pl.ds(k * ch, ch)
            xc = x_ref[rows, :]
            send_buf[rows, :] = jnp.sum(xc * xc, axis=1, keepdims=True)
            if k == 0:
                pl.semaphore_wait(barrier_sem, 1)
            rdma = pltpu.make_async_remote_copy(
                src_ref=send_buf.at[rows, :],
                dst_ref=recv_buf.at[rows, :],
                send_sem=send_sems.at[k],
                recv_sem=recv_sems.at[k],
                device_id=nbr,
                device_id_type=pl.DeviceIdType.MESH,
            )
            rdma.start()
            rdmas.append(rdma)

        g = g_ref[...]
        for k in range(_NC):
            rows = pl.ds(k * ch, ch)
            rdmas[k].wait_recv()
            totalc = send_buf[rows, :] + recv_buf[rows, :]
            invc = lax.rsqrt(totalc / n_global + 1e-5)
            o_ref[rows, :] = (x_ref[rows, :] * g * invc).astype(jnp.bfloat16)

        for k in range(_NC):
            rdmas[k].wait_send()

    return pl.pallas_call(
        body,
        out_shape=jax.ShapeDtypeStruct((m, n_local), jnp.bfloat16),
        in_specs=[
            pl.BlockSpec(memory_space=pltpu.VMEM),
            pl.BlockSpec(memory_space=pltpu.VMEM),
        ],
        out_specs=pl.BlockSpec(memory_space=pltpu.VMEM),
        scratch_shapes=[
            pltpu.VMEM((m, 1), jnp.float32),
            pltpu.VMEM((m, 1), jnp.float32),
            pltpu.SemaphoreType.DMA((_NC,)),
            pltpu.SemaphoreType.DMA((_NC,)),
        ],
        compiler_params=pltpu.CompilerParams(collective_id=0),
    )(x, gamma2)
